# baseline (speedup 1.0000x reference)
import functools

import jax
import jax.numpy as jnp
from jax import lax
from jax.experimental import pallas as pl
from jax.experimental.pallas import tpu as pltpu
from jax.experimental.pallas import tpu_sc as plsc

MAX_FRAMES = 256
DIM = 1536
LANES = 16
NBUF = 4
SUB = 2
REP = 4


@functools.cache
def _num_workers():
    try:
        info = plsc.get_sparse_core_info()
        return int(info.num_cores), int(info.num_subcores)
    except Exception:
        return 2, 16


@functools.cache
def _build(batch, seq):
    nc, ns = _num_workers()
    nw = nc * ns
    bpw = batch // nw
    mesh = plsc.VectorSubcoreMesh(core_axis_name="c", subcore_axis_name="s")

    @functools.partial(
        pl.kernel,
        mesh=mesh,
        out_type=jax.ShapeDtypeStruct((seq, batch, DIM), jnp.float32),
        scratch_types=[
            pltpu.VMEM((seq, bpw), jnp.int32),
            [pltpu.VMEM((bpw // SUB, DIM), jnp.float32) for _ in range(NBUF)],
            [pltpu.SemaphoreType.DMA for _ in range(NBUF)],
            [pltpu.SemaphoreType.DMA for _ in range(NBUF)],
        ],
    )
    def emb_kernel(table_hbm, idx_hbm, out_hbm, idx_v, rows, gsem, osem):
        wid = lax.axis_index("s") * nc + lax.axis_index("c")
        pltpu.sync_copy(idx_hbm.at[wid], idx_v)
        rep_off = (wid % REP) * MAX_FRAMES

        def clamp_row(j, carry):
            for k in range(bpw // LANES):
                v = idx_v[j, pl.ds(k * LANES, LANES)]
                v = jnp.minimum(jnp.maximum(v, 0), MAX_FRAMES - 1)
                idx_v[j, pl.ds(k * LANES, LANES)] = v + rep_off
            return carry

        lax.fori_loop(0, seq, clamp_row, 0)

        base = wid * bpw
        w = bpw // SUB
        n_sub = seq * SUB

        def src_idx(m):
            return idx_v.at[m // SUB, pl.ds((m % SUB) * w, w)]

        def out_slice(m):
            return out_hbm.at[m // SUB, pl.ds(base + (m % SUB) * w, w)]

        def start_gather(m, b):
            pltpu.async_copy(table_hbm.at[src_idx(m)], rows[b], gsem[b])

        def wait_gather(m, b):
            pltpu.make_async_copy(table_hbm.at[src_idx(m)], rows[b], gsem[b]).wait()

        def start_out(m, b):
            pltpu.async_copy(rows[b], out_slice(m), osem[b])

        def wait_out(m, b):
            pltpu.make_async_copy(rows[b], out_slice(m), osem[b]).wait()

        for b in range(NBUF):
            start_gather(b, b)

        def body(mm, carry):
            m0 = mm * NBUF
            for b in range(NBUF):
                wait_gather(m0 + b, b)
                start_out(m0 + b, b)
            for b in range(NBUF):
                wait_out(m0 + b, b)
                start_gather(m0 + NBUF + b, b)
            return carry

        lax.fori_loop(0, n_sub // NBUF - 1, body, 0)

        mlast = n_sub - NBUF
        for b in range(NBUF):
            wait_gather(mlast + b, b)
            start_out(mlast + b, b)
        for b in range(NBUF):
            wait_out(mlast + b, b)

    return emb_kernel


def kernel(frame_indices, time_emb_weight):
    batch, seq = frame_indices.shape
    nc, ns = _num_workers()
    nw = nc * ns
    bpw = batch // nw
    idx = frame_indices.astype(jnp.int32).T.reshape(seq, nw, bpw)
    idx = idx.transpose(1, 0, 2)
    table = jnp.tile(time_emb_weight, (REP, 1))
    out = _build(batch, seq)(table, idx)
    return out.transpose(1, 0, 2)


# speedup vs baseline: 1.0589x; 1.0589x over previous
import functools

import jax
import jax.numpy as jnp
from jax import lax
from jax.experimental import pallas as pl
from jax.experimental.pallas import tpu as pltpu
from jax.experimental.pallas import tpu_sc as plsc

MAX_FRAMES = 256
DIM = 1536
LANES = 16
NBUF = 8
SUB = 4
LOOKAHEAD = 4
REP = 8


@functools.cache
def _num_workers():
    try:
        info = plsc.get_sparse_core_info()
        return int(info.num_cores), int(info.num_subcores)
    except Exception:
        return 2, 16


@functools.cache
def _build(batch, seq):
    nc, ns = _num_workers()
    nw = nc * ns
    bpw = batch // nw
    mesh = plsc.VectorSubcoreMesh(core_axis_name="c", subcore_axis_name="s")

    @functools.partial(
        pl.kernel,
        mesh=mesh,
        out_type=jax.ShapeDtypeStruct((seq, batch, DIM), jnp.float32),
        scratch_types=[
            pltpu.VMEM((seq, bpw), jnp.int32),
            [pltpu.VMEM((bpw // SUB, DIM), jnp.float32) for _ in range(NBUF)],
            [pltpu.SemaphoreType.DMA for _ in range(NBUF)],
            [pltpu.SemaphoreType.DMA for _ in range(NBUF)],
        ],
    )
    def emb_kernel(table_hbm, idx_hbm, out_hbm, idx_v, rows, gsem, osem):
        wid = lax.axis_index("s") * nc + lax.axis_index("c")
        pltpu.sync_copy(idx_hbm.at[wid], idx_v)
        rep_off = (wid % REP) * MAX_FRAMES

        def clamp_row(j, carry):
            for k in range(bpw // LANES):
                v = idx_v[j, pl.ds(k * LANES, LANES)]
                v = jnp.minimum(jnp.maximum(v, 0), MAX_FRAMES - 1)
                idx_v[j, pl.ds(k * LANES, LANES)] = v + rep_off
            return carry

        lax.fori_loop(0, seq, clamp_row, 0)

        base = wid * bpw
        w = bpw // SUB
        n_sub = seq * SUB

        def src_idx(m):
            return idx_v.at[m // SUB, pl.ds((m % SUB) * w, w)]

        def out_slice(m):
            return out_hbm.at[m // SUB, pl.ds(base + (m % SUB) * w, w)]

        def start_gather(m, b):
            pltpu.async_copy(table_hbm.at[src_idx(m)], rows[b], gsem[b])

        def wait_gather(m, b):
            pltpu.make_async_copy(table_hbm.at[src_idx(m)], rows[b], gsem[b]).wait()

        def start_out(m, b):
            pltpu.async_copy(rows[b], out_slice(m), osem[b])

        def wait_out(m, b):
            pltpu.make_async_copy(rows[b], out_slice(m), osem[b]).wait()

        for m in range(LOOKAHEAD):
            start_gather(m, m % NBUF)
        for m in range(LOOKAHEAD):
            wait_gather(m, m % NBUF)
            start_out(m, m % NBUF)
            start_gather(m + LOOKAHEAD, (m + LOOKAHEAD) % NBUF)

        def body(mm, carry):
            for r in range(NBUF):
                m = LOOKAHEAD + mm * NBUF + r
                b = (LOOKAHEAD + r) % NBUF
                wait_gather(m, b)
                start_out(m, b)
                wait_out(m - LOOKAHEAD, (b - LOOKAHEAD) % NBUF)
                start_gather(m + LOOKAHEAD, (b + LOOKAHEAD) % NBUF)
            return carry

        lax.fori_loop(0, (n_sub - 2 * LOOKAHEAD) // NBUF, body, 0)

        for m in range(n_sub - LOOKAHEAD, n_sub):
            b = m % NBUF
            wait_gather(m, b)
            start_out(m, b)
            wait_out(m - LOOKAHEAD, (b - LOOKAHEAD) % NBUF)
        for m in range(n_sub - LOOKAHEAD, n_sub):
            wait_out(m, m % NBUF)

    return emb_kernel


def kernel(frame_indices, time_emb_weight):
    batch, seq = frame_indices.shape
    nc, ns = _num_workers()
    nw = nc * ns
    bpw = batch // nw
    idx = frame_indices.astype(jnp.int32).T.reshape(seq, nw, bpw)
    idx = idx.transpose(1, 0, 2)
    table = jnp.tile(time_emb_weight, (REP, 1))
    out = _build(batch, seq)(table, idx)
    return out.transpose(1, 0, 2)
